# baseline (speedup 1.0000x reference)
import functools
from functools import partial

import jax
import jax.numpy as jnp
from jax import lax
from jax.experimental import pallas as pl
from jax.experimental.pallas import tpu as pltpu
from jax.experimental.pallas import tpu_sc as plsc

_N = 10000
_E = 320000
_H = 128

_NC = 2
_NS = 16
_NW = _NC * _NS
_K = 128

_EPW = -(-_E // (_NW * _K)) * _K
_EPAD = _EPW * _NW
_CHUNKS = _EPW // _K
_NPAD = ((_N + 1 + 127) // 128) * 128
_STRIPE = _NPAD // _NS


def _sc_mesh():
    return plsc.VectorSubcoreMesh(core_axis_name="c", subcore_axis_name="s")


_NBUF = 4


def _deg_body(col_hbm, zeros_hbm, ones_hbm, out_hbm,
              col_v, ones_v, acc_sh, *sems):
    cid = lax.axis_index("c")
    sid = lax.axis_index("s")
    wid = cid * _NS + sid
    pltpu.sync_copy(zeros_hbm.at[pl.ds(sid * _STRIPE, _STRIPE)],
                    acc_sh.at[pl.ds(sid * _STRIPE, _STRIPE)])
    pltpu.sync_copy(col_hbm.at[wid], col_v)
    pltpu.sync_copy(ones_hbm, ones_v)
    plsc.subcore_barrier()

    def group(j, _):
        pltpu.sync_copy(ones_v, acc_sh.at[col_v.at[j]], add=True)
        return 0

    lax.fori_loop(0, _CHUNKS, group, 0)
    plsc.subcore_barrier()
    pltpu.sync_copy(acc_sh.at[pl.ds(sid * _STRIPE, _STRIPE)],
                    out_hbm.at[cid, pl.ds(sid * _STRIPE, _STRIPE)])


def _sc_degree(col_pad, zeros, ones):
    f = pl.kernel(
        _deg_body,
        out_type=jax.ShapeDtypeStruct((_NC, _NPAD, _H), jnp.float32),
        mesh=_sc_mesh(),
        scratch_types=[
            pltpu.VMEM((_CHUNKS, _K), jnp.int32),
            pltpu.VMEM((_K, _H), jnp.float32),
            pltpu.VMEM_SHARED((_NPAD, _H), jnp.float32),
        ] + [pltpu.SemaphoreType.DMA] * _NBUF,
    )
    return f(col_pad, zeros, ones)


_UNROLL = 4


def _agg_body(xs_hbm, row_hbm, col_hbm, zeros_hbm, out_hbm,
              row_v, col_v, buf, acc_sh, gsem):
    cid = lax.axis_index("c")
    sid = lax.axis_index("s")
    wid = cid * _NS + sid
    pltpu.sync_copy(zeros_hbm.at[pl.ds(sid * _STRIPE, _STRIPE)],
                    acc_sh.at[pl.ds(sid * _STRIPE, _STRIPE)])
    pltpu.sync_copy(row_hbm.at[pl.ds(wid * _EPW, _EPW)], row_v)
    pltpu.sync_copy(col_hbm.at[wid], col_v)
    plsc.subcore_barrier()

    def group(i, _):
        pltpu.async_copy(xs_hbm.at[row_v.at[pl.ds(i * _K, _K)]],
                         buf, gsem).wait()
        pltpu.sync_copy(buf, acc_sh.at[col_v.at[i]], add=True)
        return 0

    lax.fori_loop(0, _CHUNKS, group, 0)
    plsc.subcore_barrier()
    pltpu.sync_copy(acc_sh.at[pl.ds(sid * _STRIPE, _STRIPE)],
                    out_hbm.at[cid, pl.ds(sid * _STRIPE, _STRIPE)])


def _sc_aggregate(xs, row_pad, col_pad, zeros):
    f = pl.kernel(
        _agg_body,
        out_type=jax.ShapeDtypeStruct((_NC, _NPAD, _H), jnp.float32),
        mesh=_sc_mesh(),
        scratch_types=[
            pltpu.VMEM((_EPW,), jnp.int32),
            pltpu.VMEM((_CHUNKS, _K), jnp.int32),
            pltpu.VMEM((_K, _H), jnp.float32),
            pltpu.VMEM_SHARED((_NPAD, _H), jnp.float32),
            pltpu.SemaphoreType.DMA,
        ],
    )
    return f(xs, row_pad, col_pad, zeros)


_SQRT_HALF = 0.7071067811865476


def _gelu(t):
    return 0.5 * t * (1.0 + lax.erf(t * _SQRT_HALF))


def _ffn_body(x_ref, w1_ref, b1_ref, w2_ref, b2_ref, h_ref):
    a = _gelu(jnp.dot(x_ref[...], w1_ref[...],
                      preferred_element_type=jnp.float32) + b1_ref[...])
    h_ref[...] = _gelu(jnp.dot(a, w2_ref[...],
                               preferred_element_type=jnp.float32)
                       + b2_ref[...])


def _tc_ffn(x, w1, b1, w2, b2):
    return pl.pallas_call(
        _ffn_body,
        out_shape=jax.ShapeDtypeStruct((_N, _H), jnp.float32),
    )(x, w1, b1, w2, b2)


def _prep_body(h_ref, gw_ref, d0_ref, d1_ref, xs_ref, dinv_ref):
    deg = jnp.sum(d0_ref[...] + d1_ref[...], axis=1, keepdims=True) \
        * (1.0 / _H) + 1.0
    dinv = lax.rsqrt(deg)
    dinv_ref[...] = dinv
    xs_ref[...] = dinv * jnp.dot(h_ref[...], gw_ref[...],
                                 preferred_element_type=jnp.float32)


def _tc_prep(h, gw0, d0, d1):
    return pl.pallas_call(
        _prep_body,
        out_shape=(
            jax.ShapeDtypeStruct((_N, _H), jnp.float32),
            jax.ShapeDtypeStruct((_N, 1), jnp.float32),
        ),
    )(h, gw0, d0, d1)


def _hop_body(p0_ref, p1_ref, xs_ref, h_ref, dinv_ref, gb_ref,
              gam_ref, bet_ref, fw1_ref, fb1_ref, fw2_ref, fb2_ref,
              nw_ref, ho_ref, xso_ref):
    dinv = dinv_ref[...]
    t = dinv * (p0_ref[...] + p1_ref[...] + xs_ref[...]) + gb_ref[...] \
        + h_ref[...]
    m = jnp.mean(t, axis=0, keepdims=True)
    d = t - m
    v = jnp.mean(d * d, axis=0, keepdims=True)
    t = d * lax.rsqrt(v + 1e-5) * gam_ref[...] + bet_ref[...]
    a = _gelu(jnp.dot(t, fw1_ref[...],
                      preferred_element_type=jnp.float32) + fb1_ref[...])
    f = _gelu(jnp.dot(a, fw2_ref[...],
                      preferred_element_type=jnp.float32) + fb2_ref[...])
    h = f + t
    ho_ref[...] = h
    xso_ref[...] = dinv * jnp.dot(h, nw_ref[...],
                                  preferred_element_type=jnp.float32)


def _tc_hop(p0, p1, xs, h, dinv, gb, gam, bet, fw1, fb1, fw2, fb2, nw):
    return pl.pallas_call(
        _hop_body,
        out_shape=(
            jax.ShapeDtypeStruct((_N, _H), jnp.float32),
            jax.ShapeDtypeStruct((_N, _H), jnp.float32),
        ),
    )(p0, p1, xs, h, dinv, gb, gam, bet, fw1, fb1, fw2, fb2, nw)


def _last_body(p0_ref, p1_ref, xs_ref, h_ref, dinv_ref, gb_ref,
               gam_ref, bet_ref, fw1_ref, fb1_ref, fw2_ref, fb2_ref,
               pw1_ref, pb1_ref, pw2_ref, pb2_ref, out_ref):
    dinv = dinv_ref[...]
    t = dinv * (p0_ref[...] + p1_ref[...] + xs_ref[...]) + gb_ref[...] \
        + h_ref[...]
    m = jnp.mean(t, axis=0, keepdims=True)
    d = t - m
    v = jnp.mean(d * d, axis=0, keepdims=True)
    t = d * lax.rsqrt(v + 1e-5) * gam_ref[...] + bet_ref[...]
    a = _gelu(jnp.dot(t, fw1_ref[...],
                      preferred_element_type=jnp.float32) + fb1_ref[...])
    f = _gelu(jnp.dot(a, fw2_ref[...],
                      preferred_element_type=jnp.float32) + fb2_ref[...])
    h = f + t
    a = _gelu(jnp.dot(h, pw1_ref[...],
                      preferred_element_type=jnp.float32) + pb1_ref[...])
    out_ref[...] = _gelu(jnp.dot(a, pw2_ref[...],
                                 preferred_element_type=jnp.float32)
                         + pb2_ref[...])


def _tc_last(p0, p1, xs, h, dinv, gb, gam, bet, fw1, fb1, fw2, fb2,
             pw1, pb1, pw2, pb2):
    return pl.pallas_call(
        _last_body,
        out_shape=jax.ShapeDtypeStruct((_N, _H), jnp.float32),
    )(p0, p1, xs, h, dinv, gb, gam, bet, fw1, fb1, fw2, fb2,
      pw1, pb1, pw2, pb2)


def kernel(x, edge_index, pre_W1, pre_b1, pre_W2, pre_b2, gcn_W, gcn_b,
           bn_gamma, bn_beta, ffn_W1, ffn_b1, ffn_W2, ffn_b2,
           post_W1, post_b1, post_W2, post_b2):
    hops = gcn_W.shape[0]
    row = edge_index[0].astype(jnp.int32)
    col = edge_index[1].astype(jnp.int32)
    pad = _EPAD - _E
    row_pad = jnp.concatenate([row, jnp.zeros((pad,), jnp.int32)])
    col_pad = jnp.concatenate([col, jnp.full((pad,), _N, jnp.int32)])
    col_pad = col_pad.reshape(_NW, _CHUNKS, _K)

    zeros_nh = jnp.zeros((_NPAD, _H), jnp.float32)
    ones_k = jnp.ones((_K, _H), jnp.float32)

    deg16 = _sc_degree(col_pad, zeros_nh, ones_k)
    h = _tc_ffn(x, pre_W1, pre_b1.reshape(1, -1),
                pre_W2, pre_b2.reshape(1, -1))
    xs, dinv = _tc_prep(h, gcn_W[0], deg16[0, :_N], deg16[1, :_N])

    for i in range(hops):
        p = _sc_aggregate(xs, row_pad, col_pad, zeros_nh)
        p0 = p[0, :_N]
        p1 = p[1, :_N]
        args = (p0, p1, xs, h, dinv, gcn_b[i].reshape(1, -1),
                bn_gamma[i].reshape(1, -1), bn_beta[i].reshape(1, -1),
                ffn_W1[i], ffn_b1[i].reshape(1, -1),
                ffn_W2[i], ffn_b2[i].reshape(1, -1))
        if i + 1 < hops:
            h, xs = _tc_hop(*args, gcn_W[i + 1])
        else:
            out = _tc_last(*args, post_W1, post_b1.reshape(1, -1),
                           post_W2, post_b2.reshape(1, -1))
    return out


# speedup vs baseline: 1.0005x; 1.0005x over previous
import jax
import jax.numpy as jnp
from jax import lax
from jax.experimental import pallas as pl
from jax.experimental.pallas import tpu as pltpu
from jax.experimental.pallas import tpu_sc as plsc

_N = 10000
_E = 320000
_H = 128

_NC = 2
_NS = 16
_NW = _NC * _NS
_K = 128

_EPW = -(-_E // (_NW * _K)) * _K
_EPAD = _EPW * _NW
_CHUNKS = _EPW // _K
_NPAD = ((_N + 1 + 127) // 128) * 128
_STRIPE = _NPAD // _NS


def _sc_mesh():
    return plsc.VectorSubcoreMesh(core_axis_name="c", subcore_axis_name="s")


def _deg_body(col_hbm, zeros_hbm, ones_hbm, out_hbm,
              col_v, ones_v, acc_sh):
    cid = lax.axis_index("c")
    sid = lax.axis_index("s")
    wid = cid * _NS + sid
    pltpu.sync_copy(zeros_hbm.at[pl.ds(sid * _STRIPE, _STRIPE)],
                    acc_sh.at[pl.ds(sid * _STRIPE, _STRIPE)])
    pltpu.sync_copy(col_hbm.at[wid], col_v)
    pltpu.sync_copy(ones_hbm, ones_v)
    plsc.subcore_barrier()

    def group(j, _):
        pltpu.sync_copy(ones_v, acc_sh.at[col_v.at[j]], add=True)
        return 0

    lax.fori_loop(0, _CHUNKS, group, 0)
    plsc.subcore_barrier()
    pltpu.sync_copy(acc_sh.at[pl.ds(sid * _STRIPE, _STRIPE)],
                    out_hbm.at[cid, pl.ds(sid * _STRIPE, _STRIPE)])


def _sc_degree(col_pad, zeros, ones):
    f = pl.kernel(
        _deg_body,
        out_type=jax.ShapeDtypeStruct((_NC, _NPAD, _H), jnp.float32),
        mesh=_sc_mesh(),
        scratch_types=[
            pltpu.VMEM((_CHUNKS, _K), jnp.int32),
            pltpu.VMEM((_K, _H), jnp.float32),
            pltpu.VMEM_SHARED((_NPAD, _H), jnp.float32),
        ],
    )
    return f(col_pad, zeros, ones)


def _agg_body(xs_hbm, row_hbm, col_hbm, zeros_hbm, out_hbm,
              row_v, col_v, buf, acc_sh, gsem):
    cid = lax.axis_index("c")
    sid = lax.axis_index("s")
    wid = cid * _NS + sid
    pltpu.sync_copy(zeros_hbm.at[pl.ds(sid * _STRIPE, _STRIPE)],
                    acc_sh.at[pl.ds(sid * _STRIPE, _STRIPE)])
    pltpu.sync_copy(row_hbm.at[pl.ds(wid * _EPW, _EPW)], row_v)
    pltpu.sync_copy(col_hbm.at[wid], col_v)
    plsc.subcore_barrier()

    def group(i, _):
        pltpu.async_copy(xs_hbm.at[row_v.at[pl.ds(i * _K, _K)]],
                         buf, gsem).wait()
        pltpu.sync_copy(buf, acc_sh.at[col_v.at[i]], add=True)
        return 0

    lax.fori_loop(0, _CHUNKS, group, 0)
    plsc.subcore_barrier()
    pltpu.sync_copy(acc_sh.at[pl.ds(sid * _STRIPE, _STRIPE)],
                    out_hbm.at[cid, pl.ds(sid * _STRIPE, _STRIPE)])


def _sc_aggregate(xs, row_pad, col_pad, zeros):
    f = pl.kernel(
        _agg_body,
        out_type=jax.ShapeDtypeStruct((_NC, _NPAD, _H), jnp.float32),
        mesh=_sc_mesh(),
        scratch_types=[
            pltpu.VMEM((_EPW,), jnp.int32),
            pltpu.VMEM((_CHUNKS, _K), jnp.int32),
            pltpu.VMEM((_K, _H), jnp.float32),
            pltpu.VMEM_SHARED((_NPAD, _H), jnp.float32),
            pltpu.SemaphoreType.DMA,
        ],
    )
    return f(xs, row_pad, col_pad, zeros)


_SQRT_HALF = 0.7071067811865476


def _gelu(t):
    return 0.5 * t * (1.0 + lax.erf(t * _SQRT_HALF))


def _ffn_body(x_ref, w1_ref, b1_ref, w2_ref, b2_ref, h_ref):
    a = _gelu(jnp.dot(x_ref[...], w1_ref[...],
                      preferred_element_type=jnp.float32) + b1_ref[...])
    h_ref[...] = _gelu(jnp.dot(a, w2_ref[...],
                               preferred_element_type=jnp.float32)
                       + b2_ref[...])


def _tc_ffn(x, w1, b1, w2, b2):
    return pl.pallas_call(
        _ffn_body,
        out_shape=jax.ShapeDtypeStruct((_N, _H), jnp.float32),
    )(x, w1, b1, w2, b2)


def _prep_body(h_ref, gw_ref, d0_ref, d1_ref, xs_ref, dinv_ref):
    deg = jnp.sum(d0_ref[...] + d1_ref[...], axis=1, keepdims=True) \
        * (1.0 / _H) + 1.0
    dinv = lax.rsqrt(deg)
    dinv_ref[...] = dinv
    xs_ref[...] = dinv * jnp.dot(h_ref[...], gw_ref[...],
                                 preferred_element_type=jnp.float32)


def _tc_prep(h, gw0, d0, d1):
    return pl.pallas_call(
        _prep_body,
        out_shape=(
            jax.ShapeDtypeStruct((_N, _H), jnp.float32),
            jax.ShapeDtypeStruct((_N, 1), jnp.float32),
        ),
    )(h, gw0, d0, d1)


def _hop_body(p0_ref, p1_ref, xs_ref, h_ref, dinv_ref, gb_ref,
              gam_ref, bet_ref, fw1_ref, fb1_ref, fw2_ref, fb2_ref,
              nw_ref, ho_ref, xso_ref):
    dinv = dinv_ref[...]
    t = dinv * (p0_ref[...] + p1_ref[...] + xs_ref[...]) + gb_ref[...] \
        + h_ref[...]
    m = jnp.mean(t, axis=0, keepdims=True)
    d = t - m
    v = jnp.mean(d * d, axis=0, keepdims=True)
    t = d * lax.rsqrt(v + 1e-5) * gam_ref[...] + bet_ref[...]
    a = _gelu(jnp.dot(t, fw1_ref[...],
                      preferred_element_type=jnp.float32) + fb1_ref[...])
    f = _gelu(jnp.dot(a, fw2_ref[...],
                      preferred_element_type=jnp.float32) + fb2_ref[...])
    h = f + t
    ho_ref[...] = h
    xso_ref[...] = dinv * jnp.dot(h, nw_ref[...],
                                  preferred_element_type=jnp.float32)


def _tc_hop(p0, p1, xs, h, dinv, gb, gam, bet, fw1, fb1, fw2, fb2, nw):
    return pl.pallas_call(
        _hop_body,
        out_shape=(
            jax.ShapeDtypeStruct((_N, _H), jnp.float32),
            jax.ShapeDtypeStruct((_N, _H), jnp.float32),
        ),
    )(p0, p1, xs, h, dinv, gb, gam, bet, fw1, fb1, fw2, fb2, nw)


def _last_body(p0_ref, p1_ref, xs_ref, h_ref, dinv_ref, gb_ref,
               gam_ref, bet_ref, fw1_ref, fb1_ref, fw2_ref, fb2_ref,
               pw1_ref, pb1_ref, pw2_ref, pb2_ref, out_ref):
    dinv = dinv_ref[...]
    t = dinv * (p0_ref[...] + p1_ref[...] + xs_ref[...]) + gb_ref[...] \
        + h_ref[...]
    m = jnp.mean(t, axis=0, keepdims=True)
    d = t - m
    v = jnp.mean(d * d, axis=0, keepdims=True)
    t = d * lax.rsqrt(v + 1e-5) * gam_ref[...] + bet_ref[...]
    a = _gelu(jnp.dot(t, fw1_ref[...],
                      preferred_element_type=jnp.float32) + fb1_ref[...])
    f = _gelu(jnp.dot(a, fw2_ref[...],
                      preferred_element_type=jnp.float32) + fb2_ref[...])
    h = f + t
    a = _gelu(jnp.dot(h, pw1_ref[...],
                      preferred_element_type=jnp.float32) + pb1_ref[...])
    out_ref[...] = _gelu(jnp.dot(a, pw2_ref[...],
                                 preferred_element_type=jnp.float32)
                         + pb2_ref[...])


def _tc_last(p0, p1, xs, h, dinv, gb, gam, bet, fw1, fb1, fw2, fb2,
             pw1, pb1, pw2, pb2):
    return pl.pallas_call(
        _last_body,
        out_shape=jax.ShapeDtypeStruct((_N, _H), jnp.float32),
    )(p0, p1, xs, h, dinv, gb, gam, bet, fw1, fb1, fw2, fb2,
      pw1, pb1, pw2, pb2)


def kernel(x, edge_index, pre_W1, pre_b1, pre_W2, pre_b2, gcn_W, gcn_b,
           bn_gamma, bn_beta, ffn_W1, ffn_b1, ffn_W2, ffn_b2,
           post_W1, post_b1, post_W2, post_b2):
    hops = gcn_W.shape[0]
    row = edge_index[0].astype(jnp.int32)
    col = edge_index[1].astype(jnp.int32)
    pad = _EPAD - _E
    row_pad = jnp.concatenate([row, jnp.zeros((pad,), jnp.int32)])
    col_pad = jnp.concatenate([col, jnp.full((pad,), _N, jnp.int32)])
    col_pad = col_pad.reshape(_NW, _CHUNKS, _K)

    zeros_nh = jnp.zeros((_NPAD, _H), jnp.float32)
    ones_k = jnp.ones((_K, _H), jnp.float32)

    deg16 = _sc_degree(col_pad, zeros_nh, ones_k)
    h = _tc_ffn(x, pre_W1, pre_b1.reshape(1, -1),
                pre_W2, pre_b2.reshape(1, -1))
    xs, dinv = _tc_prep(h, gcn_W[0], deg16[0, :_N], deg16[1, :_N])

    for i in range(hops):
        p = _sc_aggregate(xs, row_pad, col_pad, zeros_nh)
        p0 = p[0, :_N]
        p1 = p[1, :_N]
        args = (p0, p1, xs, h, dinv, gcn_b[i].reshape(1, -1),
                bn_gamma[i].reshape(1, -1), bn_beta[i].reshape(1, -1),
                ffn_W1[i], ffn_b1[i].reshape(1, -1),
                ffn_W2[i], ffn_b2[i].reshape(1, -1))
        if i + 1 < hops:
            h, xs = _tc_hop(*args, gcn_W[i + 1])
        else:
            out = _tc_last(*args, post_W1, post_b1.reshape(1, -1),
                           post_W2, post_b2.reshape(1, -1))
    return out
